# baseline (speedup 1.0000x reference)
import functools

import jax
import jax.numpy as jnp
from jax import lax
from jax.experimental import pallas as pl
from jax.experimental.pallas import tpu as pltpu
from jax.experimental.pallas import tpu_sc as plsc

B = 16384
C = 1000
FILL = 1.0 / (C - 1)

NC = 2
NS = 16
NW = NC * NS
CPW = B // NW
CB = 128
NJ = CPW // CB
QROWS = (248, 248, 248, 256)
QBASE = (0, 248, 496, 744)
L = 16


@functools.partial(
    pl.kernel,
    mesh=plsc.VectorSubcoreMesh(core_axis_name="c", subcore_axis_name="s"),
    out_type=jax.ShapeDtypeStruct((C, B), jnp.float32),
    scratch_types=[
        pltpu.VMEM((QROWS[0], CB), jnp.float32),
        pltpu.VMEM((QROWS[1], CB), jnp.float32),
        pltpu.VMEM((QROWS[2], CB), jnp.float32),
        pltpu.VMEM((QROWS[3], CB), jnp.float32),
        pltpu.VMEM((CPW,), jnp.int32),
        pltpu.SemaphoreType.DMA,
        pltpu.SemaphoreType.DMA,
        pltpu.SemaphoreType.DMA,
        pltpu.SemaphoreType.DMA,
    ],
)
def _virtual_teacher(y_hbm, out_hbm, bq0, bq1, bq2, bq3, yv,
                     sm0, sm1, sm2, sm3):
    wid = lax.axis_index("s") * NC + lax.axis_index("c")
    base = wid * CPW

    bufs = (bq0, bq1, bq2, bq3)
    sems = (sm0, sm1, sm2, sm3)

    pltpu.sync_copy(y_hbm.at[pl.ds(base, CPW)], yv)

    fill_vec = jnp.full((L,), FILL, dtype=jnp.float32)
    iota = lax.iota(jnp.int32, L)

    def fill(q):
        buf = bufs[q]

        def fill_row(r, carry):
            for k in range(CB // L):
                buf[r, pl.ds(k * L, L)] = fill_vec
            return carry

        lax.fori_loop(0, QROWS[q], fill_row, 0)

    def scan_pass(q, j, value):
        buf, h0, hrows = bufs[q], QBASE[q], QROWS[q]

        def group(g, carry):
            ys = yv[pl.ds(j * CB + g * L, L)]
            cstart = pl.multiple_of(g * L, L)
            for jj in range(L):
                y_r = ys[jj]
                hit = jnp.logical_and(y_r >= h0, y_r < h0 + hrows)
                row = jnp.clip(y_r - h0, 0, hrows - 1)
                lane = jnp.where(hit, jj, -1)
                old = buf[row, pl.ds(cstart, L)]
                buf[row, pl.ds(cstart, L)] = jnp.where(iota == lane, value, old)
            return carry

        lax.fori_loop(0, CB // L, group, 0)

    def fire(q, j):
        return pltpu.async_copy(
            bufs[q],
            out_hbm.at[
                pl.ds(QBASE[q], QROWS[q]), pl.ds(base + j * CB, CB)
            ],
            sems[q],
        )

    def drain(q):
        pltpu.make_async_copy(
            bufs[q],
            out_hbm.at[pl.ds(QBASE[q], QROWS[q]), pl.ds(base, CB)],
            sems[q],
        ).wait()

    for q in range(4):
        fill(q)
        scan_pass(q, 0, 0.0)
        fire(q, 0)

    def chunk_body(j, carry):
        for q in range(4):
            drain(q)
            scan_pass(q, j - 1, FILL)
            scan_pass(q, j, 0.0)
            fire(q, j)
        return carry

    lax.fori_loop(1, NJ, chunk_body, 0)

    for q in range(4):
        drain(q)


def kernel(x, y):
    del x
    return _virtual_teacher(y.astype(jnp.int32)).T


# speedup vs baseline: 1.1251x; 1.1251x over previous
import functools

import jax
import jax.numpy as jnp
from jax import lax
from jax.experimental import pallas as pl
from jax.experimental.pallas import tpu as pltpu
from jax.experimental.pallas import tpu_sc as plsc

B = 16384
C = 1000
FILL = 1.0 / (C - 1)

NC = 2
NS = 16
NW = NC * NS
CPW = B // NW
CB = 128
NJ = CPW // CB
HA = 496
HB = C - HA
L = 16


@functools.partial(
    pl.kernel,
    mesh=plsc.VectorSubcoreMesh(core_axis_name="c", subcore_axis_name="s"),
    out_type=jax.ShapeDtypeStruct((C, B), jnp.float32),
    scratch_types=[
        pltpu.VMEM((HA, CB), jnp.float32),
        pltpu.VMEM((HB, CB), jnp.float32),
        pltpu.VMEM((CPW,), jnp.int32),
        pltpu.SemaphoreType.DMA,
        pltpu.SemaphoreType.DMA,
    ],
)
def _virtual_teacher(y_hbm, out_hbm, buf_a, buf_b, yv, sem_a, sem_b):
    wid = lax.axis_index("s") * NC + lax.axis_index("c")
    base = wid * CPW

    pltpu.sync_copy(y_hbm.at[pl.ds(base, CPW)], yv)

    fill_vec = jnp.full((L,), FILL, dtype=jnp.float32)
    iota = lax.iota(jnp.int32, L)

    def fill_a(r, carry):
        for k in range(CB // L):
            buf_a[r, pl.ds(k * L, L)] = fill_vec
        return carry

    def fill_b(r, carry):
        for k in range(CB // L):
            buf_b[r, pl.ds(k * L, L)] = fill_vec
        return carry

    def scan_pass(buf, h0, hrows, j, value):
        def group(g, carry):
            ys = yv[pl.ds(j * CB + g * L, L)]
            cstart = pl.multiple_of(g * L, L)
            for jj in range(L):
                y_r = ys[jj]
                hit = jnp.logical_and(y_r >= h0, y_r < h0 + hrows)
                row = jnp.clip(y_r - h0, 0, hrows - 1)
                lane = jnp.where(hit, jj, -1)
                old = buf[row, pl.ds(cstart, L)]
                buf[row, pl.ds(cstart, L)] = jnp.where(iota == lane, value, old)
            return carry

        lax.fori_loop(0, CB // L, group, 0)

    def fire(buf, h0, j, sem):
        return pltpu.async_copy(
            buf,
            out_hbm.at[pl.ds(h0, buf.shape[0]), pl.ds(base + j * CB, CB)],
            sem,
        )

    def drain(buf, h0, sem):
        pltpu.make_async_copy(
            buf, out_hbm.at[pl.ds(h0, buf.shape[0]), pl.ds(base, CB)], sem
        ).wait()

    lax.fori_loop(0, HA, fill_a, 0)
    scan_pass(buf_a, 0, HA, 0, 0.0)
    fire(buf_a, 0, 0, sem_a)
    lax.fori_loop(0, HB, fill_b, 0)
    scan_pass(buf_b, HA, HB, 0, 0.0)
    fire(buf_b, HA, 0, sem_b)

    def chunk_body(j, carry):
        drain(buf_a, 0, sem_a)
        scan_pass(buf_a, 0, HA, j - 1, FILL)
        scan_pass(buf_a, 0, HA, j, 0.0)
        fire(buf_a, 0, j, sem_a)
        drain(buf_b, HA, sem_b)
        scan_pass(buf_b, HA, HB, j - 1, FILL)
        scan_pass(buf_b, HA, HB, j, 0.0)
        fire(buf_b, HA, j, sem_b)
        return carry

    lax.fori_loop(1, NJ, chunk_body, 0)

    drain(buf_a, 0, sem_a)
    drain(buf_b, HA, sem_b)


def kernel(x, y):
    del x
    return _virtual_teacher(y.astype(jnp.int32)).T


# speedup vs baseline: 1.1389x; 1.0123x over previous
import functools

import jax
import jax.numpy as jnp
from jax import lax
from jax.experimental import pallas as pl
from jax.experimental.pallas import tpu as pltpu
from jax.experimental.pallas import tpu_sc as plsc

B = 16384
C = 1000
FILL = 1.0 / (C - 1)

NC = 2
NS = 16
NW = NC * NS
CPW = B // NW
CB = 128
NJ = CPW // CB
HA = 496
HB = C - HA
L = 16


@functools.partial(
    pl.kernel,
    mesh=plsc.VectorSubcoreMesh(core_axis_name="c", subcore_axis_name="s"),
    out_type=jax.ShapeDtypeStruct((C, B), jnp.float32),
    scratch_types=[
        pltpu.VMEM((HA, CB), jnp.float32),
        pltpu.VMEM((HB, CB), jnp.float32),
        pltpu.VMEM((CPW,), jnp.int32),
        pltpu.SemaphoreType.DMA,
        pltpu.SemaphoreType.DMA,
    ],
)
def _virtual_teacher(y_hbm, out_hbm, buf_a, buf_b, yv, sem_a, sem_b):
    wid = lax.axis_index("s") * NC + lax.axis_index("c")
    base = wid * CPW

    y_copy = pltpu.async_copy(y_hbm.at[pl.ds(base, CPW)], yv, sem_b)

    fill_vec = jnp.full((L,), FILL, dtype=jnp.float32)
    iota = lax.iota(jnp.int32, L)

    def fill_a(r, carry):
        for k in range(CB // L):
            buf_a[r, pl.ds(k * L, L)] = fill_vec
        return carry

    def fill_b(r, carry):
        for k in range(CB // L):
            buf_b[r, pl.ds(k * L, L)] = fill_vec
        return carry

    def scan_range(buf, b0, r0, rlen, j, value):
        nrows = buf.shape[0]

        def group(g, carry):
            ys = yv[pl.ds(j * CB + g * L, L)]
            cstart = pl.multiple_of(g * L, L)
            rows_v = jnp.clip(ys - b0, 0, nrows - 1)
            hits_v = (ys - r0).astype(jnp.uint32) < rlen
            lanes_v = jnp.where(hits_v, iota, -1)
            for jj in range(L):
                old = buf[rows_v[jj], pl.ds(cstart, L)]
                buf[rows_v[jj], pl.ds(cstart, L)] = jnp.where(
                    iota == lanes_v[jj], value, old
                )
            return carry

        lax.fori_loop(0, CB // L, group, 0)

    def scan_pass(buf, h0, hrows, j, value):
        scan_range(buf, h0, h0, hrows, j, value)

    def fire(buf, h0, j, sem):
        return pltpu.async_copy(
            buf,
            out_hbm.at[pl.ds(h0, buf.shape[0]), pl.ds(base + j * CB, CB)],
            sem,
        )

    def drain(buf, h0, sem):
        pltpu.make_async_copy(
            buf, out_hbm.at[pl.ds(h0, buf.shape[0]), pl.ds(base, CB)], sem
        ).wait()

    HA2 = HA // 2
    lax.fori_loop(0, HA2, fill_a, 0)
    y_copy.wait()
    scan_range(buf_a, 0, 0, HA2, 0, 0.0)
    pltpu.async_copy(
        buf_a.at[pl.ds(0, HA2)],
        out_hbm.at[pl.ds(0, HA2), pl.ds(base, CB)],
        sem_a,
    )
    lax.fori_loop(HA2, HA, fill_a, 0)
    scan_range(buf_a, 0, HA2, HA - HA2, 0, 0.0)
    pltpu.async_copy(
        buf_a.at[pl.ds(HA2, HA - HA2)],
        out_hbm.at[pl.ds(HA2, HA - HA2), pl.ds(base, CB)],
        sem_a,
    )
    lax.fori_loop(0, HB, fill_b, 0)
    scan_pass(buf_b, HA, HB, 0, 0.0)
    fire(buf_b, HA, 0, sem_b)

    def chunk_body(j, carry):
        drain(buf_a, 0, sem_a)
        scan_pass(buf_a, 0, HA, j - 1, FILL)
        scan_pass(buf_a, 0, HA, j, 0.0)
        fire(buf_a, 0, j, sem_a)
        drain(buf_b, HA, sem_b)
        scan_pass(buf_b, HA, HB, j - 1, FILL)
        scan_pass(buf_b, HA, HB, j, 0.0)
        fire(buf_b, HA, j, sem_b)
        return carry

    lax.fori_loop(1, NJ, chunk_body, 0)

    drain(buf_a, 0, sem_a)
    drain(buf_b, HA, sem_b)


def kernel(x, y):
    del x
    return _virtual_teacher(y.astype(jnp.int32)).T


# speedup vs baseline: 1.1753x; 1.0319x over previous
import functools

import jax
import jax.numpy as jnp
from jax import lax
from jax.experimental import pallas as pl
from jax.experimental.pallas import tpu as pltpu
from jax.experimental.pallas import tpu_sc as plsc

B = 16384
C = 1000
FILL = 1.0 / (C - 1)

NC = 2
NS = 16
NW = NC * NS
CPW = B // NW
CB = 128
NJ = CPW // CB
HA = 496
HB = C - HA
L = 16


@functools.partial(
    pl.kernel,
    mesh=plsc.VectorSubcoreMesh(core_axis_name="c", subcore_axis_name="s"),
    out_type=jax.ShapeDtypeStruct((C, B), jnp.float32),
    scratch_types=[
        pltpu.VMEM((HA, CB), jnp.float32),
        pltpu.VMEM((HB, CB), jnp.float32),
        pltpu.VMEM((CPW,), jnp.int32),
        pltpu.SemaphoreType.DMA,
        pltpu.SemaphoreType.DMA,
    ],
)
def _virtual_teacher(y_hbm, out_hbm, buf_a, buf_b, yv, sem_a, sem_b):
    wid = lax.axis_index("s") * NC + lax.axis_index("c")
    base = wid * CPW

    y_copy = pltpu.async_copy(y_hbm.at[pl.ds(base, CPW)], yv, sem_b)

    fill_vec = jnp.full((L,), FILL, dtype=jnp.float32)
    iota = lax.iota(jnp.int32, L)

    def fill_a(r, carry):
        for k in range(CB // L):
            buf_a[r, pl.ds(k * L, L)] = fill_vec
        return carry

    def fill_b(r, carry):
        for k in range(CB // L):
            buf_b[r, pl.ds(k * L, L)] = fill_vec
        return carry

    def scan_range(buf, b0, r0, rlen, j, value):
        nrows = buf.shape[0]

        def group(g, carry):
            ys = yv[pl.ds(j * CB + g * L, L)]
            cstart = pl.multiple_of(g * L, L)
            rows_v = jnp.clip(ys - b0, 0, nrows - 1)
            hits_v = (ys - r0).astype(jnp.uint32) < rlen
            lanes_v = jnp.where(hits_v, iota, -1)
            for jj in range(L):
                old = buf[rows_v[jj], pl.ds(cstart, L)]
                buf[rows_v[jj], pl.ds(cstart, L)] = jnp.where(
                    iota == lanes_v[jj], value, old
                )
            return carry

        lax.fori_loop(0, CB // L, group, 0)

    def scan_pass(buf, h0, hrows, j, value):
        scan_range(buf, h0, h0, hrows, j, value)

    def fire(buf, h0, j, sem):
        return pltpu.async_copy(
            buf,
            out_hbm.at[pl.ds(h0, buf.shape[0]), pl.ds(base + j * CB, CB)],
            sem,
        )

    def drain(buf, h0, sem):
        pltpu.make_async_copy(
            buf, out_hbm.at[pl.ds(h0, buf.shape[0]), pl.ds(base, CB)], sem
        ).wait()

    lax.fori_loop(0, HA, fill_a, 0)
    y_copy.wait()
    scan_pass(buf_a, 0, HA, 0, 0.0)
    fire(buf_a, 0, 0, sem_a)
    lax.fori_loop(0, HB, fill_b, 0)
    scan_pass(buf_b, HA, HB, 0, 0.0)
    fire(buf_b, HA, 0, sem_b)

    def chunk_body(j, carry):
        drain(buf_a, 0, sem_a)
        scan_pass(buf_a, 0, HA, j - 1, FILL)
        scan_pass(buf_a, 0, HA, j, 0.0)
        fire(buf_a, 0, j, sem_a)
        drain(buf_b, HA, sem_b)
        scan_pass(buf_b, HA, HB, j - 1, FILL)
        scan_pass(buf_b, HA, HB, j, 0.0)
        fire(buf_b, HA, j, sem_b)
        return carry

    lax.fori_loop(1, NJ, chunk_body, 0)

    drain(buf_a, 0, sem_a)
    drain(buf_b, HA, sem_b)


def kernel(x, y):
    del x
    return _virtual_teacher(y.astype(jnp.int32)).T
